# baseline (speedup 1.0000x reference)
import functools

import jax
import jax.numpy as jnp
from jax.experimental import pallas as pl
from jax.experimental.pallas import tpu as pltpu

_BLOCK_ROWS = 4000


def _fused_linear_kernel(x_ref, wc_ref, bc_ref, wb_ref, bb_ref,
                         scores_ref, deltas_ref):
    x = x_ref[...]
    scores_ref[...] = (
        jnp.dot(x, wc_ref[...], preferred_element_type=jnp.float32)
        + bc_ref[...]
    )
    deltas_ref[...] = (
        jnp.dot(x, wb_ref[...], preferred_element_type=jnp.float32)
        + bb_ref[...]
    )


@jax.jit
def kernel(x, W_cls, b_cls, W_box, b_box):
    if x.ndim > 2:
        x = x.reshape(x.shape[0], -1)
    n, in_dim = x.shape
    n_cls = W_cls.shape[0]
    n_box = W_box.shape[0]

    wc_t = W_cls.T
    wb_t = W_box.T
    bc = b_cls.reshape(1, n_cls)
    bb = b_box.reshape(1, n_box)

    grid = (pl.cdiv(n, _BLOCK_ROWS),)
    scores, deltas = pl.pallas_call(
        _fused_linear_kernel,
        grid=grid,
        in_specs=[
            pl.BlockSpec((_BLOCK_ROWS, in_dim), lambda i: (i, 0)),
            pl.BlockSpec((in_dim, n_cls), lambda i: (0, 0)),
            pl.BlockSpec((1, n_cls), lambda i: (0, 0)),
            pl.BlockSpec((in_dim, n_box), lambda i: (0, 0)),
            pl.BlockSpec((1, n_box), lambda i: (0, 0)),
        ],
        out_specs=[
            pl.BlockSpec((_BLOCK_ROWS, n_cls), lambda i: (i, 0)),
            pl.BlockSpec((_BLOCK_ROWS, n_box), lambda i: (i, 0)),
        ],
        out_shape=[
            jax.ShapeDtypeStruct((n, n_cls), jnp.float32),
            jax.ShapeDtypeStruct((n, n_box), jnp.float32),
        ],
        compiler_params=pltpu.CompilerParams(
            dimension_semantics=("parallel",),
        ),
    )(x, wc_t, bc, wb_t, bb)
    return (scores, deltas)


# speedup vs baseline: 1.0125x; 1.0125x over previous
import jax
import jax.numpy as jnp
from jax.experimental import pallas as pl
from jax.experimental.pallas import tpu as pltpu

_BLOCK_ROWS = 2000


def _fused_linear_kernel(xl_ref, xh_ref, wcl_ref, wch_ref, bc_ref,
                         wbl_ref, wbh_ref, bb_ref, scores_ref, deltas_ref):
    xl = xl_ref[...]
    xh = xh_ref[...]
    scores_ref[...] = (
        jnp.dot(xl, wcl_ref[...], preferred_element_type=jnp.float32)
        + jnp.dot(xh, wch_ref[...], preferred_element_type=jnp.float32)
        + bc_ref[...]
    )
    deltas_ref[...] = (
        jnp.dot(xl, wbl_ref[...], preferred_element_type=jnp.float32)
        + jnp.dot(xh, wbh_ref[...], preferred_element_type=jnp.float32)
        + bb_ref[...]
    )


@jax.jit
def kernel(x, W_cls, b_cls, W_box, b_box):
    if x.ndim > 2:
        x = x.reshape(x.shape[0], -1)
    n, in_dim = x.shape
    n_cls = W_cls.shape[0]
    n_box = W_box.shape[0]
    half = in_dim // 2

    wc_t = W_cls.T
    wb_t = W_box.T
    bc = b_cls.reshape(1, n_cls)
    bb = b_box.reshape(1, n_box)

    grid = (pl.cdiv(n, _BLOCK_ROWS),)
    scores, deltas = pl.pallas_call(
        _fused_linear_kernel,
        grid=grid,
        in_specs=[
            pl.BlockSpec((_BLOCK_ROWS, half), lambda i: (i, 0)),
            pl.BlockSpec((_BLOCK_ROWS, half), lambda i: (i, 1)),
            pl.BlockSpec((half, n_cls), lambda i: (0, 0)),
            pl.BlockSpec((half, n_cls), lambda i: (1, 0)),
            pl.BlockSpec((1, n_cls), lambda i: (0, 0)),
            pl.BlockSpec((half, n_box), lambda i: (0, 0)),
            pl.BlockSpec((half, n_box), lambda i: (1, 0)),
            pl.BlockSpec((1, n_box), lambda i: (0, 0)),
        ],
        out_specs=[
            pl.BlockSpec((_BLOCK_ROWS, n_cls), lambda i: (i, 0)),
            pl.BlockSpec((_BLOCK_ROWS, n_box), lambda i: (i, 0)),
        ],
        out_shape=[
            jax.ShapeDtypeStruct((n, n_cls), jnp.float32),
            jax.ShapeDtypeStruct((n, n_box), jnp.float32),
        ],
        compiler_params=pltpu.CompilerParams(
            dimension_semantics=("arbitrary",),
        ),
    )(x, x, wc_t, wc_t, bc, wb_t, wb_t, bb)
    return (scores, deltas)


# speedup vs baseline: 1.9619x; 1.9377x over previous
import jax
import jax.numpy as jnp
from jax.experimental import pallas as pl
from jax.experimental.pallas import tpu as pltpu

_BLOCK_ROWS = 2048

_DN = (((1,), (1,)), ((), ()))


def _fused_linear_kernel(x_ref, wc_ref, bc_ref, wb_ref, bb_ref,
                         scores_t_ref, deltas_t_ref):
    x = x_ref[...]
    scores_t_ref[...] = (
        jax.lax.dot_general(wc_ref[...], x, _DN,
                            preferred_element_type=jnp.float32)
        + bc_ref[...]
    )
    deltas_t_ref[...] = (
        jax.lax.dot_general(wb_ref[...], x, _DN,
                            preferred_element_type=jnp.float32)
        + bb_ref[...]
    )


@jax.jit
def kernel(x, W_cls, b_cls, W_box, b_box):
    if x.ndim > 2:
        x = x.reshape(x.shape[0], -1)
    n, in_dim = x.shape
    n_cls = W_cls.shape[0]
    n_box = W_box.shape[0]

    bc = b_cls.reshape(n_cls, 1)
    bb = b_box.reshape(n_box, 1)

    grid = (pl.cdiv(n, _BLOCK_ROWS),)
    scores_t, deltas_t = pl.pallas_call(
        _fused_linear_kernel,
        grid=grid,
        in_specs=[
            pl.BlockSpec((_BLOCK_ROWS, in_dim), lambda i: (i, 0)),
            pl.BlockSpec((n_cls, in_dim), lambda i: (0, 0)),
            pl.BlockSpec((n_cls, 1), lambda i: (0, 0)),
            pl.BlockSpec((n_box, in_dim), lambda i: (0, 0)),
            pl.BlockSpec((n_box, 1), lambda i: (0, 0)),
        ],
        out_specs=[
            pl.BlockSpec((n_cls, _BLOCK_ROWS), lambda i: (0, i)),
            pl.BlockSpec((n_box, _BLOCK_ROWS), lambda i: (0, i)),
        ],
        out_shape=[
            jax.ShapeDtypeStruct((n_cls, n), jnp.float32),
            jax.ShapeDtypeStruct((n_box, n), jnp.float32),
        ],
        compiler_params=pltpu.CompilerParams(
            dimension_semantics=("arbitrary",),
        ),
    )(x, W_cls, bc, W_box, bb)
    return (scores_t.T, deltas_t.T)
